# baseline (speedup 1.0000x reference)
import jax
import jax.numpy as jnp
from jax import lax
from jax.experimental import pallas as pl
from jax.experimental.pallas import tpu as pltpu, tpu_sc as plsc

_INPUT_LEN = 200
_EMBED_DIM = 64
_BATCH = 4096
_FLAT = _INPUT_LEN * _EMBED_DIM

_B_TC = 2048
_B_SC = _BATCH - _B_TC

_NC = 2
_NS = 16
_NW = _NC * _NS
_ROWS_PER_W = _B_SC // _NW
_REP = 8
_BLOCKS_PER_W = _ROWS_PER_W // _REP

_TR = 256
_NB_TC = _B_TC // _TR


def _make_sc_kernel():
    mesh = plsc.VectorSubcoreMesh(core_axis_name="c", subcore_axis_name="s")

    @pl.kernel(
        mesh=mesh,
        out_type=jax.ShapeDtypeStruct((_BATCH, _FLAT), jnp.float32),
        scratch_types=[
            pltpu.VMEM((_REP, _FLAT), jnp.float32),
            pltpu.SemaphoreType.DMA,
        ],
    )
    def sc_kernel(pos_hbm, out_hbm, tile_v, sem):
        wid = lax.axis_index("s") * _NC + lax.axis_index("c")
        base = _B_TC + wid * _ROWS_PER_W
        fills = [pltpu.async_copy(pos_hbm, tile_v.at[r], sem) for r in range(_REP)]
        for f in fills:
            f.wait()
        outs = [
            pltpu.async_copy(
                tile_v, out_hbm.at[pl.ds(base + j * _REP, _REP), :], sem
            )
            for j in range(_BLOCKS_PER_W)
        ]
        for c in outs:
            c.wait()

    return sc_kernel


_SC_KERNEL = _make_sc_kernel()


def _tc_body(flat_ref, pos_ref, out_ref, tile_ref, sem):
    del flat_ref
    tile_ref[...] = jnp.broadcast_to(pos_ref[...], tile_ref.shape)
    copies = [
        pltpu.make_async_copy(tile_ref, out_ref.at[pl.ds(j * _TR, _TR), :], sem)
        for j in range(_NB_TC)
    ]
    for c in copies:
        c.start()
    for c in copies:
        c.wait()


def kernel(x, pos_table):
    del x
    pos_flat = pos_table.reshape(1, _FLAT)
    half = _SC_KERNEL(pos_flat.reshape(_FLAT))
    out = pl.pallas_call(
        _tc_body,
        in_specs=[
            pl.BlockSpec(memory_space=pl.ANY),
            pl.BlockSpec((1, _FLAT), lambda: (0, 0)),
        ],
        out_specs=pl.BlockSpec(memory_space=pl.ANY),
        out_shape=jax.ShapeDtypeStruct((_BATCH, _FLAT), jnp.float32),
        input_output_aliases={0: 0},
        scratch_shapes=[
            pltpu.VMEM((_TR, _FLAT), jnp.float32),
            pltpu.SemaphoreType.DMA,
        ],
    )(half, pos_flat)
    return out.reshape(_BATCH, _INPUT_LEN, _EMBED_DIM)
